# Initial kernel scaffold; baseline (speedup 1.0000x reference)
import jax
import jax.numpy as jnp
from jax.experimental import pallas as pl


def kernel(x, edge_index, edge_weight, edge_weight_multiplier, neuron_threshold, fc_w, fc_b):
    raise NotImplementedError("write your pallas kernel here")


# speedup vs baseline: 179.6740x; 179.6740x over previous
import functools

import jax
import jax.numpy as jnp
from jax import lax
from jax.experimental import pallas as pl
from jax.experimental.pallas import tpu as pltpu
from jax.experimental.pallas import tpu_sc as plsc

B = 16
N = 10000
E = 320000
NN = B * N
BE = B * E
NUM_PASSES = 4

CH = 128
CPW = 10
K = CH * CPW
NWIN = (BE // 32) // K


def _sc_pass_body(x_hbm, src_hbm, dst_hbm, ew_hbm, ewm_hbm, out_hbm,
                  x_sp, aggr_sp,
                  zbuf, src_win, dst_win, ew_win, ewm_win, xs_win, msg_win,
                  sem_lin, sem_g, sem_s, sem_x):
    c = lax.axis_index("c")
    s = lax.axis_index("s")

    if True:
        my_off = c * 80000 + s * 5000

        def zero_body(i, _):
            zbuf[pl.ds(i * 16, 16)] = jnp.zeros((16,), jnp.float32)
            return 0
        lax.fori_loop(0, 313, zero_body, 0)

        pltpu.async_copy(zbuf.at[pl.ds(0, 5000)],
                         aggr_sp.at[pl.ds(my_off, 5000)], sem_x).wait()
        pltpu.async_copy(x_hbm.at[pl.ds(my_off, 5000)],
                         zbuf.at[pl.ds(0, 5000)], sem_x).wait()
        pltpu.async_copy(zbuf.at[pl.ds(0, 5000)],
                         x_sp.at[pl.ds(my_off, 5000)], sem_x).wait()
        plsc.subcore_barrier()

        r0 = c * 20000 + s * 1250
        h = s % 2
        we0 = h * 1250

        def window(k, _):
            cps = [
                pltpu.async_copy(src_hbm.at[pl.ds(r0 + k * CPW, CPW)],
                                 src_win, sem_lin),
                pltpu.async_copy(dst_hbm.at[pl.ds(r0 + k * CPW, CPW)],
                                 dst_win, sem_lin),
                pltpu.async_copy(ew_hbm.at[pl.ds(we0 + k * CPW, CPW)],
                                 ew_win, sem_lin),
                pltpu.async_copy(ewm_hbm.at[pl.ds(we0 + k * CPW, CPW)],
                                 ewm_win, sem_lin),
            ]
            for cp in cps:
                cp.wait()
            gcps = [pltpu.async_copy(x_sp.at[src_win.at[j, 0]],
                                     xs_win.at[j, 0], sem_g)
                    for j in range(CPW)]
            for cp in gcps:
                cp.wait()

            def mul_body(i, _):
                row = i // 8
                col = (i % 8) * 16
                m = (xs_win[row, 0, pl.ds(col, 16)]
                     * ew_win[row, 0, pl.ds(col, 16)]
                     * ewm_win[row, 0, pl.ds(col, 16)])
                msg_win[row, 0, pl.ds(col, 16)] = m
                return 0
            lax.fori_loop(0, K // 16, mul_body, 0)

            scps = [pltpu.async_copy(msg_win.at[j, 0],
                                     aggr_sp.at[dst_win.at[j, 0]], sem_s,
                                     add=True)
                    for j in range(CPW)]
            for cp in scps:
                cp.wait()
            return 0

        lax.fori_loop(0, NWIN, window, 0)
        plsc.subcore_barrier()

        pltpu.async_copy(aggr_sp.at[pl.ds(my_off, 5000)],
                         zbuf.at[pl.ds(0, 5000)], sem_x).wait()
        pltpu.async_copy(zbuf.at[pl.ds(0, 5000)],
                         out_hbm.at[pl.ds(my_off, 5000)], sem_x).wait()


def _sc_pass(xf, src2d, dst2d, ew2d, ewm2d):
    mesh = plsc.VectorSubcoreMesh(core_axis_name="c", subcore_axis_name="s")
    f = functools.partial(
        pl.kernel,
        out_type=jax.ShapeDtypeStruct((NN,), jnp.float32),
        mesh=mesh,
        scratch_types=[
            pltpu.VMEM_SHARED((NN,), jnp.float32),
            pltpu.VMEM_SHARED((NN,), jnp.float32),
            pltpu.VMEM((5008,), jnp.float32),
            pltpu.VMEM((CPW, 1, CH), jnp.int32),
            pltpu.VMEM((CPW, 1, CH), jnp.int32),
            pltpu.VMEM((CPW, 1, CH), jnp.float32),
            pltpu.VMEM((CPW, 1, CH), jnp.float32),
            pltpu.VMEM((CPW, 1, CH), jnp.float32),
            pltpu.VMEM((CPW, 1, CH), jnp.float32),
            pltpu.SemaphoreType.DMA,
            pltpu.SemaphoreType.DMA,
            pltpu.SemaphoreType.DMA,
            pltpu.SemaphoreType.DMA,
        ],
    )(_sc_pass_body)
    return f(xf, src2d, dst2d, ew2d, ewm2d)


def kernel(x, edge_index, edge_weight, edge_weight_multiplier,
           neuron_threshold, fc_w, fc_b):
    xf = x[:, 0]
    src2d = edge_index[0].reshape(-1, 1, CH)
    dst2d = edge_index[1].reshape(-1, 1, CH)
    ew2d = edge_weight.reshape(-1, 1, CH)
    ewm2d = edge_weight_multiplier.reshape(-1, 1, CH)
    thr = jnp.abs(neuron_threshold)
    for _ in range(NUM_PASSES):
        aggr = _sc_pass(xf, src2d, dst2d, ew2d, ewm2d)
        t = (aggr - aggr.min()) / (aggr.max() - aggr.min())
        xf = jax.nn.sigmoid(t.reshape(B, N) - thr[None, :]).reshape(-1)
    xm = jnp.mean(xf.reshape(B, N)[:, ::10], axis=1)
    return xm[:, None] * fc_w[:, 0][None, :] + fc_b[None, :]


# speedup vs baseline: 336.8044x; 1.8745x over previous
import functools

import jax
import jax.numpy as jnp
from jax import lax
from jax.experimental import pallas as pl
from jax.experimental.pallas import tpu as pltpu
from jax.experimental.pallas import tpu_sc as plsc

B = 16
N = 10000
E = 320000
NN = B * N
BE = B * E
NUM_PASSES = 4

CH = 128
CPW = 25
K = CH * CPW
NWIN = (BE // 32) // K


def _sc_pass_body(x_hbm, src_hbm, dst_hbm, wc_hbm, out_hbm,
                  x_sp, aggr_sp, zbuf,
                  src0, src1, dst0, dst1, w0, w1, xs0, xs1, msg0, msg1,
                  semL0, semL1, semD0, semD1, sem_g, semS0, semS1, sem_x):
    c = lax.axis_index("c")
    s = lax.axis_index("s")

    my_off = c * 80000 + s * 5000
    r0 = c * 20000 + s * 1250
    we0 = (s % 2) * 1250

    def zero_body(i, _):
        zbuf[pl.ds(i * 16, 16)] = jnp.zeros((16,), jnp.float32)
        return 0
    lax.fori_loop(0, 313, zero_body, 0)

    pltpu.async_copy(zbuf.at[pl.ds(0, 5000)],
                     aggr_sp.at[pl.ds(my_off, 5000)], sem_x).wait()
    pltpu.async_copy(x_hbm.at[pl.ds(my_off, 5000)],
                     zbuf.at[pl.ds(0, 5000)], sem_x).wait()
    pltpu.async_copy(zbuf.at[pl.ds(0, 5000)],
                     x_sp.at[pl.ds(my_off, 5000)], sem_x).wait()
    plsc.subcore_barrier()

    def window(k, srcb, dstb, wb, xsb, msgb, semL, semD, semS,
               first, prefetch):
        rb = r0 + k * CPW
        wrb = we0 + k * CPW
        pltpu.make_async_copy(src_hbm.at[pl.ds(r0, CPW)], srcb, semL).wait()
        pltpu.make_async_copy(wc_hbm.at[pl.ds(we0, CPW)], wb, semL).wait()
        for j in range(CPW):
            pltpu.async_copy(x_sp.at[srcb.at[j, 0]], xsb.at[j, 0], sem_g)
        if not first:
            pltpu.make_async_copy(wc_hbm.at[pl.ds(we0, CPW)], msgb,
                                  semS).wait()
        pltpu.async_copy(dst_hbm.at[pl.ds(rb, CPW)], dstb, semD)
        pltpu.make_async_copy(wc_hbm.at[pl.ds(we0, CPW)], xsb, sem_g).wait()

        def fire_src():
            pltpu.async_copy(src_hbm.at[pl.ds(rb + 2 * CPW, CPW)], srcb,
                             semL)
        if prefetch is True:
            fire_src()
        elif prefetch is not False:
            pl.when(prefetch)(fire_src)

        def row_body(r, _):
            for cc in range(CH // 16):
                col = cc * 16
                msgb[r, 0, pl.ds(col, 16)] = (
                    xsb[r, 0, pl.ds(col, 16)] * wb[r, 0, pl.ds(col, 16)])
            return 0
        lax.fori_loop(0, CPW, row_body, 0)

        def fire_w():
            pltpu.async_copy(wc_hbm.at[pl.ds(wrb + 2 * CPW, CPW)], wb, semL)
        if prefetch is True:
            fire_w()
        elif prefetch is not False:
            pl.when(prefetch)(fire_w)

        pltpu.make_async_copy(dst_hbm.at[pl.ds(rb, CPW)], dstb, semD).wait()
        for j in range(CPW):
            pltpu.async_copy(msgb.at[j, 0], aggr_sp.at[dstb.at[j, 0]], semS,
                             add=True)

    pltpu.async_copy(src_hbm.at[pl.ds(r0, CPW)], src0, semL0)
    pltpu.async_copy(wc_hbm.at[pl.ds(we0, CPW)], w0, semL0)
    pltpu.async_copy(src_hbm.at[pl.ds(r0 + CPW, CPW)], src1, semL1)
    pltpu.async_copy(wc_hbm.at[pl.ds(we0 + CPW, CPW)], w1, semL1)

    window(0, src0, dst0, w0, xs0, msg0, semL0, semD0, semS0, True, True)
    window(1, src1, dst1, w1, xs1, msg1, semL1, semD1, semS1, True, True)

    def pair_body(m, _):
        pf = m < (NWIN // 2 - 1)
        window(2 * m, src0, dst0, w0, xs0, msg0, semL0, semD0, semS0,
               False, pf)
        window(2 * m + 1, src1, dst1, w1, xs1, msg1, semL1, semD1, semS1,
               False, pf)
        return 0
    lax.fori_loop(1, NWIN // 2, pair_body, 0)

    pltpu.make_async_copy(wc_hbm.at[pl.ds(we0, CPW)], msg0, semS0).wait()
    pltpu.make_async_copy(wc_hbm.at[pl.ds(we0, CPW)], msg1, semS1).wait()
    plsc.subcore_barrier()

    pltpu.async_copy(aggr_sp.at[pl.ds(my_off, 5000)],
                     zbuf.at[pl.ds(0, 5000)], sem_x).wait()
    pltpu.async_copy(zbuf.at[pl.ds(0, 5000)],
                     out_hbm.at[pl.ds(my_off, 5000)], sem_x).wait()


def _sc_pass(xf, src2d, dst2d, wc2d):
    mesh = plsc.VectorSubcoreMesh(core_axis_name="c", subcore_axis_name="s")
    f = functools.partial(
        pl.kernel,
        out_type=jax.ShapeDtypeStruct((NN,), jnp.float32),
        mesh=mesh,
        scratch_types=[
            pltpu.VMEM_SHARED((NN,), jnp.float32),
            pltpu.VMEM_SHARED((NN,), jnp.float32),
            pltpu.VMEM((5008,), jnp.float32),
            pltpu.VMEM((CPW, 1, CH), jnp.int32),
            pltpu.VMEM((CPW, 1, CH), jnp.int32),
            pltpu.VMEM((CPW, 1, CH), jnp.int32),
            pltpu.VMEM((CPW, 1, CH), jnp.int32),
            pltpu.VMEM((CPW, 1, CH), jnp.float32),
            pltpu.VMEM((CPW, 1, CH), jnp.float32),
            pltpu.VMEM((CPW, 1, CH), jnp.float32),
            pltpu.VMEM((CPW, 1, CH), jnp.float32),
            pltpu.VMEM((CPW, 1, CH), jnp.float32),
            pltpu.VMEM((CPW, 1, CH), jnp.float32),
            pltpu.SemaphoreType.DMA,
            pltpu.SemaphoreType.DMA,
            pltpu.SemaphoreType.DMA,
            pltpu.SemaphoreType.DMA,
            pltpu.SemaphoreType.DMA,
            pltpu.SemaphoreType.DMA,
            pltpu.SemaphoreType.DMA,
            pltpu.SemaphoreType.DMA,
        ],
    )(_sc_pass_body)
    return f(xf, src2d, dst2d, wc2d)


def kernel(x, edge_index, edge_weight, edge_weight_multiplier,
           neuron_threshold, fc_w, fc_b):
    xf = x[:, 0]
    src2d = edge_index[0].reshape(-1, 1, CH)
    dst2d = edge_index[1].reshape(-1, 1, CH)
    wc2d = (edge_weight * edge_weight_multiplier).reshape(-1, 1, CH)
    thr = jnp.abs(neuron_threshold)
    for _ in range(NUM_PASSES):
        aggr = _sc_pass(xf, src2d, dst2d, wc2d)
        t = (aggr - aggr.min()) / (aggr.max() - aggr.min())
        xf = jax.nn.sigmoid(t.reshape(B, N) - thr[None, :]).reshape(-1)
    xm = jnp.mean(xf.reshape(B, N)[:, ::10], axis=1)
    return xm[:, None] * fc_w[:, 0][None, :] + fc_b[None, :]


# speedup vs baseline: 350.7731x; 1.0415x over previous
import functools

import jax
import jax.numpy as jnp
from jax import lax
from jax.experimental import pallas as pl
from jax.experimental.pallas import tpu as pltpu
from jax.experimental.pallas import tpu_sc as plsc

B = 16
N = 10000
E = 320000
NN = B * N
BE = B * E
NUM_PASSES = 4

K = 3200
NWIN = (BE // 32) // K


def _sc_pass_body(x_hbm, src_hbm, dst_hbm, wc_hbm, out_hbm,
                  x_sp, aggr_sp, zbuf,
                  src0, src1, dst0, dst1, w0, w1, xs0, xs1, msg0, msg1,
                  semL0, semL1, semD0, semD1, sem_g, semS0, semS1, sem_x):
    c = lax.axis_index("c")
    s = lax.axis_index("s")

    my_off = c * 80000 + s * 5000
    e0 = c * (BE // 2) + s * (BE // 32)
    wf0 = (s % 2) * (E // 2)

    def zero_body(i, _):
        zbuf[pl.ds(i * 16, 16)] = jnp.zeros((16,), jnp.float32)
        return 0
    lax.fori_loop(0, 313, zero_body, 0)

    pltpu.async_copy(zbuf.at[pl.ds(0, 5000)],
                     aggr_sp.at[pl.ds(my_off, 5000)], sem_x).wait()
    pltpu.async_copy(x_hbm.at[pl.ds(my_off, 5000)],
                     zbuf.at[pl.ds(0, 5000)], sem_x).wait()
    pltpu.async_copy(zbuf.at[pl.ds(0, 5000)],
                     x_sp.at[pl.ds(my_off, 5000)], sem_x).wait()
    plsc.subcore_barrier()

    def window(k, srcb, dstb, wb, xsb, msgb, semL, semD, semS,
               first, prefetch):
        eb = e0 + k * K
        wbse = wf0 + k * K
        pltpu.make_async_copy(src_hbm.at[pl.ds(e0, K)], srcb, semL).wait()
        pltpu.make_async_copy(wc_hbm.at[pl.ds(wf0, K)], wb, semL).wait()
        pltpu.async_copy(x_sp.at[srcb], xsb, sem_g)
        if not first:
            pltpu.make_async_copy(wc_hbm.at[pl.ds(wf0, K)], msgb,
                                  semS).wait()
        pltpu.async_copy(dst_hbm.at[pl.ds(eb, K)], dstb, semD)
        pltpu.make_async_copy(wc_hbm.at[pl.ds(wf0, K)], xsb, sem_g).wait()

        def fire_src():
            pltpu.async_copy(src_hbm.at[pl.ds(eb + 2 * K, K)], srcb, semL)
        if prefetch is True:
            fire_src()
        elif prefetch is not False:
            pl.when(prefetch)(fire_src)

        def vec_body(i, _):
            msgb[pl.ds(i * 16, 16)] = (xsb[pl.ds(i * 16, 16)]
                                       * wb[pl.ds(i * 16, 16)])
            return 0
        lax.fori_loop(0, K // 16, vec_body, 0)

        def fire_w():
            pltpu.async_copy(wc_hbm.at[pl.ds(wbse + 2 * K, K)], wb, semL)
        if prefetch is True:
            fire_w()
        elif prefetch is not False:
            pl.when(prefetch)(fire_w)

        pltpu.make_async_copy(dst_hbm.at[pl.ds(eb, K)], dstb, semD).wait()
        pltpu.async_copy(msgb, aggr_sp.at[dstb], semS, add=True)

    pltpu.async_copy(src_hbm.at[pl.ds(e0, K)], src0, semL0)
    pltpu.async_copy(wc_hbm.at[pl.ds(wf0, K)], w0, semL0)
    pltpu.async_copy(src_hbm.at[pl.ds(e0 + K, K)], src1, semL1)
    pltpu.async_copy(wc_hbm.at[pl.ds(wf0 + K, K)], w1, semL1)

    window(0, src0, dst0, w0, xs0, msg0, semL0, semD0, semS0, True, True)
    window(1, src1, dst1, w1, xs1, msg1, semL1, semD1, semS1, True, True)

    def pair_body(m, _):
        pf = m < (NWIN // 2 - 1)
        window(2 * m, src0, dst0, w0, xs0, msg0, semL0, semD0, semS0,
               False, pf)
        window(2 * m + 1, src1, dst1, w1, xs1, msg1, semL1, semD1, semS1,
               False, pf)
        return 0
    lax.fori_loop(1, NWIN // 2, pair_body, 0)

    pltpu.make_async_copy(wc_hbm.at[pl.ds(wf0, K)], msg0, semS0).wait()
    pltpu.make_async_copy(wc_hbm.at[pl.ds(wf0, K)], msg1, semS1).wait()
    plsc.subcore_barrier()

    pltpu.async_copy(aggr_sp.at[pl.ds(my_off, 5000)],
                     zbuf.at[pl.ds(0, 5000)], sem_x).wait()
    pltpu.async_copy(zbuf.at[pl.ds(0, 5000)],
                     out_hbm.at[pl.ds(my_off, 5000)], sem_x).wait()


def _sc_pass(xf, src1d, dst1d, wc1d):
    mesh = plsc.VectorSubcoreMesh(core_axis_name="c", subcore_axis_name="s")
    f = functools.partial(
        pl.kernel,
        out_type=jax.ShapeDtypeStruct((NN,), jnp.float32),
        mesh=mesh,
        scratch_types=[
            pltpu.VMEM_SHARED((NN,), jnp.float32),
            pltpu.VMEM_SHARED((NN,), jnp.float32),
            pltpu.VMEM((5008,), jnp.float32),
            pltpu.VMEM((K,), jnp.int32),
            pltpu.VMEM((K,), jnp.int32),
            pltpu.VMEM((K,), jnp.int32),
            pltpu.VMEM((K,), jnp.int32),
            pltpu.VMEM((K,), jnp.float32),
            pltpu.VMEM((K,), jnp.float32),
            pltpu.VMEM((K,), jnp.float32),
            pltpu.VMEM((K,), jnp.float32),
            pltpu.VMEM((K,), jnp.float32),
            pltpu.VMEM((K,), jnp.float32),
            pltpu.SemaphoreType.DMA,
            pltpu.SemaphoreType.DMA,
            pltpu.SemaphoreType.DMA,
            pltpu.SemaphoreType.DMA,
            pltpu.SemaphoreType.DMA,
            pltpu.SemaphoreType.DMA,
            pltpu.SemaphoreType.DMA,
            pltpu.SemaphoreType.DMA,
        ],
    )(_sc_pass_body)
    return f(xf, src1d, dst1d, wc1d)


def kernel(x, edge_index, edge_weight, edge_weight_multiplier,
           neuron_threshold, fc_w, fc_b):
    xf = x[:, 0]
    src1d = edge_index[0]
    dst1d = edge_index[1]
    wc1d = edge_weight * edge_weight_multiplier
    thr = jnp.abs(neuron_threshold)
    for _ in range(NUM_PASSES):
        aggr = _sc_pass(xf, src1d, dst1d, wc1d)
        t = (aggr - aggr.min()) / (aggr.max() - aggr.min())
        xf = jax.nn.sigmoid(t.reshape(B, N) - thr[None, :]).reshape(-1)
    xm = jnp.mean(xf.reshape(B, N)[:, ::10], axis=1)
    return xm[:, None] * fc_w[:, 0][None, :] + fc_b[None, :]
